# baseline (speedup 1.0000x reference)
import jax
import jax.numpy as jnp
from jax import lax
from jax.experimental import pallas as pl
from jax.experimental.pallas import tpu as pltpu

_H = 768
_E = 64
_B = 32768
_BLK = 2048


def _dot(a, b):
    return lax.dot_general(a, b, (((1,), (0,)), ((), ())),
                           preferred_element_type=jnp.float32)


def _main_body(x_ref, wvt_ref, wot_ref, rw1t_ref, rw2t_ref, bv_ref, bo_ref,
               logits_ref, idx_ref, w_ref, lv_ref, ent_ref,
               ls_ref, load_acc, ent_acc):
    i = pl.program_id(0)
    nblk = pl.num_programs(0) - 1
    cur = lax.rem(i, 2)

    @pl.when(i == 0)
    def _init():
        load_acc[...] = jnp.zeros_like(load_acc)
        ent_acc[...] = jnp.zeros_like(ent_acc)

    @pl.when(i < nblk)
    def _matmul():
        x16 = x_ref[...].astype(jnp.bfloat16)
        v = _dot(x16, wvt_ref[...]) + bv_ref[...]
        a = _dot(v.astype(jnp.bfloat16), wot_ref[...]) + bo_ref[...]
        logits = _dot(x16, rw1t_ref[...]) + _dot(a.astype(jnp.bfloat16),
                                                 rw2t_ref[...])
        logits_ref[...] = logits
        ls_ref[pl.ds(cur * _BLK, _BLK), :] = logits

    @pl.when(i > 0)
    def _stats():
        l = ls_ref[pl.ds((1 - cur) * _BLK, _BLK), :]
        m1 = jnp.max(l, axis=-1, keepdims=True)
        d = l - m1
        e = jnp.exp(d)
        s = jnp.sum(e, axis=-1, keepdims=True)
        rs = 1.0 / s
        p = e * rs

        load_acc[...] += jnp.sum(p, axis=0, keepdims=True)
        plogp = jnp.sum(e * d, axis=-1, keepdims=True) * rs - jnp.log(s)
        ent_acc[...] += jnp.sum(plogp, axis=0, keepdims=True)

        colf = lax.broadcasted_iota(jnp.int32, l.shape, 1).astype(jnp.float32)
        big = jnp.float32(_E)
        i1f = jnp.min(jnp.where(l == m1, colf, big), axis=-1, keepdims=True)
        masked = jnp.where(colf == i1f, -jnp.inf, l)
        m2 = jnp.max(masked, axis=-1, keepdims=True)
        i2f = jnp.min(jnp.where(masked == m2, colf, big), axis=-1,
                      keepdims=True)
        idx_ref[...] = jnp.concatenate([i1f, i2f], axis=1).astype(jnp.int32)

        t = jnp.exp(m2 - m1)
        w1 = 1.0 / (1.0 + t)
        w_ref[...] = jnp.concatenate([w1, 1.0 - w1], axis=1)

    @pl.when(i == nblk)
    def _finalize():
        el = load_acc[...] / _B
        mu = jnp.mean(el)
        lv_ref[...] = jnp.sum((el - mu) ** 2, keepdims=True)[:, :1] / (_E - 1)
        ent_ref[...] = -ent_acc[...] / _B


def kernel(hidden_states, enc_w1, enc_b1, ln_g, ln_b, enc_w2, enc_b2,
           in_proj_w, in_proj_b, out_proj_w, out_proj_b, router_w):
    wvt = in_proj_w[2 * _H:].T.astype(jnp.bfloat16)
    wot = out_proj_w.T.astype(jnp.bfloat16)
    rw1t = router_w[:, :_H].T.astype(jnp.bfloat16)
    rw2t = router_w[:, _H:].T.astype(jnp.bfloat16)
    bv = in_proj_b[2 * _H:].reshape(1, _H)
    bo = out_proj_b.reshape(1, _H)

    nblk = _B // _BLK
    logits, idx, w, lv, ent = pl.pallas_call(
        _main_body,
        grid=(nblk + 1,),
        in_specs=[
            pl.BlockSpec((_BLK, _H), lambda i: (jnp.minimum(i, nblk - 1), 0)),
            pl.BlockSpec((_H, _H), lambda i: (0, 0)),
            pl.BlockSpec((_H, _H), lambda i: (0, 0)),
            pl.BlockSpec((_H, _E), lambda i: (0, 0)),
            pl.BlockSpec((_H, _E), lambda i: (0, 0)),
            pl.BlockSpec((1, _H), lambda i: (0, 0)),
            pl.BlockSpec((1, _H), lambda i: (0, 0)),
        ],
        out_specs=[
            pl.BlockSpec((_BLK, _E), lambda i: (jnp.minimum(i, nblk - 1), 0)),
            pl.BlockSpec((_BLK, 2), lambda i: (jnp.maximum(i - 1, 0), 0)),
            pl.BlockSpec((_BLK, 2), lambda i: (jnp.maximum(i - 1, 0), 0)),
            pl.BlockSpec((1, 1), lambda i: (0, 0)),
            pl.BlockSpec((1, 1), lambda i: (0, 0)),
        ],
        out_shape=[
            jax.ShapeDtypeStruct((_B, _E), jnp.float32),
            jax.ShapeDtypeStruct((_B, 2), jnp.int32),
            jax.ShapeDtypeStruct((_B, 2), jnp.float32),
            jax.ShapeDtypeStruct((1, 1), jnp.float32),
            jax.ShapeDtypeStruct((1, 1), jnp.float32),
        ],
        scratch_shapes=[
            pltpu.VMEM((2 * _BLK, _E), jnp.float32),
            pltpu.VMEM((1, _E), jnp.float32),
            pltpu.VMEM((1, 1), jnp.float32),
        ],
        compiler_params=pltpu.CompilerParams(
            dimension_semantics=("arbitrary",)),
    )(hidden_states, wvt, wot, rw1t, rw2t, bv, bo)

    return (logits, idx, w, lv.reshape(()), ent.reshape(()))


# speedup vs baseline: 1.0915x; 1.0915x over previous
import jax
import jax.numpy as jnp
from jax import lax
from jax.experimental import pallas as pl
from jax.experimental.pallas import tpu as pltpu

_H = 768
_E = 64
_B = 32768
_BLK = 2048


def _dot(a, b):
    return lax.dot_general(a, b, (((1,), (0,)), ((), ())),
                           preferred_element_type=jnp.float32)


def _main_body(x_ref, wvt_ref, wot_ref, rw1t_ref, rw2t_ref, bv_ref, bo_ref,
               logits_ref, idx_ref, w_ref, lv_ref, ent_ref,
               ls_ref, load_acc, ent_acc):
    i = pl.program_id(0)
    nblk = pl.num_programs(0) - 1
    first = i == 0

    l = ls_ref[...]
    m1 = jnp.max(l, axis=-1, keepdims=True)
    d = l - m1
    e = jnp.exp(d)
    s = jnp.sum(e, axis=-1, keepdims=True)
    rs = 1.0 / s
    p = e * rs

    load_c = jnp.where(first, 0.0, jnp.sum(p, axis=0, keepdims=True))
    load_acc[...] = jnp.where(first, 0.0, load_acc[...]) + load_c
    plogp = jnp.sum(e * d, axis=-1, keepdims=True) * rs - jnp.log(s)
    ent_c = jnp.where(first, 0.0, jnp.sum(plogp, axis=0, keepdims=True))
    ent_acc[...] = jnp.where(first, 0.0, ent_acc[...]) + ent_c

    colf = lax.broadcasted_iota(jnp.int32, l.shape, 1).astype(jnp.float32)
    big = jnp.float32(_E)
    i1f = jnp.min(jnp.where(l == m1, colf, big), axis=-1, keepdims=True)
    masked = jnp.where(colf == i1f, -jnp.inf, l)
    m2 = jnp.max(masked, axis=-1, keepdims=True)
    i2f = jnp.min(jnp.where(masked == m2, colf, big), axis=-1, keepdims=True)
    idx_ref[...] = jnp.concatenate([i1f, i2f], axis=1).astype(jnp.int32)

    t = jnp.exp(m2 - m1)
    w1 = 1.0 / (1.0 + t)
    w_ref[...] = jnp.concatenate([w1, 1.0 - w1], axis=1)

    x16 = x_ref[...].astype(jnp.bfloat16)
    v = _dot(x16, wvt_ref[...]) + bv_ref[...]
    a = _dot(v.astype(jnp.bfloat16), wot_ref[...]) + bo_ref[...]
    logits = _dot(x16, rw1t_ref[...]) + _dot(a.astype(jnp.bfloat16),
                                             rw2t_ref[...])
    logits_ref[...] = logits
    ls_ref[...] = logits

    @pl.when(i == nblk)
    def _finalize():
        el = load_acc[...] / _B
        mu = jnp.mean(el)
        lv_ref[...] = jnp.sum((el - mu) ** 2, keepdims=True)[:, :1] / (_E - 1)
        ent_ref[...] = -ent_acc[...] / _B


def kernel(hidden_states, enc_w1, enc_b1, ln_g, ln_b, enc_w2, enc_b2,
           in_proj_w, in_proj_b, out_proj_w, out_proj_b, router_w):
    wvt = in_proj_w[2 * _H:].T.astype(jnp.bfloat16)
    wot = out_proj_w.T.astype(jnp.bfloat16)
    rw1t = router_w[:, :_H].T.astype(jnp.bfloat16)
    rw2t = router_w[:, _H:].T.astype(jnp.bfloat16)
    bv = in_proj_b[2 * _H:].reshape(1, _H)
    bo = out_proj_b.reshape(1, _H)

    nblk = _B // _BLK
    logits, idx, w, lv, ent = pl.pallas_call(
        _main_body,
        grid=(nblk + 1,),
        in_specs=[
            pl.BlockSpec((_BLK, _H), lambda i: (jnp.minimum(i, nblk - 1), 0)),
            pl.BlockSpec((_H, _H), lambda i: (0, 0)),
            pl.BlockSpec((_H, _H), lambda i: (0, 0)),
            pl.BlockSpec((_H, _E), lambda i: (0, 0)),
            pl.BlockSpec((_H, _E), lambda i: (0, 0)),
            pl.BlockSpec((1, _H), lambda i: (0, 0)),
            pl.BlockSpec((1, _H), lambda i: (0, 0)),
        ],
        out_specs=[
            pl.BlockSpec((_BLK, _E), lambda i: (jnp.minimum(i, nblk - 1), 0)),
            pl.BlockSpec((_BLK, 2), lambda i: (jnp.maximum(i - 1, 0), 0)),
            pl.BlockSpec((_BLK, 2), lambda i: (jnp.maximum(i - 1, 0), 0)),
            pl.BlockSpec((1, 1), lambda i: (0, 0)),
            pl.BlockSpec((1, 1), lambda i: (0, 0)),
        ],
        out_shape=[
            jax.ShapeDtypeStruct((_B, _E), jnp.float32),
            jax.ShapeDtypeStruct((_B, 2), jnp.int32),
            jax.ShapeDtypeStruct((_B, 2), jnp.float32),
            jax.ShapeDtypeStruct((1, 1), jnp.float32),
            jax.ShapeDtypeStruct((1, 1), jnp.float32),
        ],
        scratch_shapes=[
            pltpu.VMEM((_BLK, _E), jnp.float32),
            pltpu.VMEM((1, _E), jnp.float32),
            pltpu.VMEM((1, 1), jnp.float32),
        ],
        compiler_params=pltpu.CompilerParams(
            dimension_semantics=("arbitrary",)),
    )(hidden_states, wvt, wot, rw1t, rw2t, bv, bo)

    return (logits, idx, w, lv.reshape(()), ent.reshape(()))


# speedup vs baseline: 1.0917x; 1.0002x over previous
import jax
import jax.numpy as jnp
from jax import lax
from jax.experimental import pallas as pl
from jax.experimental.pallas import tpu as pltpu

_H = 768
_E = 64
_B = 32768
_BLK = 2048


def _dot(a, b):
    return lax.dot_general(a, b, (((1,), (0,)), ((), ())),
                           preferred_element_type=jnp.float32)


def _main_body(x_ref, wvt_ref, wot_ref, rw1t_ref, rw2t_ref, bv_ref, bo_ref,
               logits_ref, idx_ref, w_ref, lv_ref, ent_ref,
               ls_ref, load_acc, ent_acc):
    i = pl.program_id(0)
    nblk = pl.num_programs(0) - 1
    first = i == 0

    l = ls_ref[...]
    m1 = jnp.max(l, axis=-1, keepdims=True)
    d = l - m1
    e = jnp.exp(d)
    s = jnp.sum(e, axis=-1, keepdims=True)
    rs = 1.0 / s
    p = e * rs

    load_c = jnp.where(first, 0.0, jnp.sum(p, axis=0, keepdims=True))
    load_acc[...] = jnp.where(first, 0.0, load_acc[...]) + load_c
    plogp = jnp.sum(e * d, axis=-1, keepdims=True) * rs - jnp.log(s)
    ent_c = jnp.where(first, 0.0, jnp.sum(plogp, axis=0, keepdims=True))
    ent_acc[...] = jnp.where(first, 0.0, ent_acc[...]) + ent_c

    colf = lax.broadcasted_iota(jnp.int32, l.shape, 1).astype(jnp.float32)
    big = jnp.float32(_E)
    i1f = jnp.min(jnp.where(l == m1, colf, big), axis=-1, keepdims=True)
    masked = jnp.where(colf == i1f, -jnp.inf, l)
    m2 = jnp.max(masked, axis=-1, keepdims=True)
    i2f = jnp.min(jnp.where(masked == m2, colf, big), axis=-1, keepdims=True)
    idx_ref[...] = jnp.concatenate([i1f, i2f], axis=1).astype(jnp.int32)

    t = jnp.exp(m2 - m1)
    w1 = 1.0 / (1.0 + t)
    w_ref[...] = jnp.concatenate([w1, 1.0 - w1], axis=1)

    x16 = x_ref[...].astype(jnp.bfloat16)
    v = _dot(x16, wvt_ref[...]) + bv_ref[...]
    a = _dot(v.astype(jnp.bfloat16), wot_ref[...]) + bo_ref[...]
    logits = _dot(x16, rw1t_ref[...]) + _dot(a.astype(jnp.bfloat16),
                                             rw2t_ref[...])
    logits_ref[...] = logits
    ls_ref[...] = logits

    @pl.when(i == nblk)
    def _finalize():
        el = load_acc[...] / _B
        mu = jnp.mean(el)
        lv_ref[...] = jnp.sum((el - mu) ** 2, keepdims=True)[:, :1] / (_E - 1)
        ent_ref[...] = -ent_acc[...] / _B


def kernel(hidden_states, enc_w1, enc_b1, ln_g, ln_b, enc_w2, enc_b2,
           in_proj_w, in_proj_b, out_proj_w, out_proj_b, router_w):
    wvt = in_proj_w[2 * _H:].T.astype(jnp.bfloat16)
    wot = out_proj_w.T.astype(jnp.bfloat16)
    rw1t = router_w[:, :_H].T.astype(jnp.bfloat16)
    rw2t = router_w[:, _H:].T.astype(jnp.bfloat16)
    bv = in_proj_b[2 * _H:].reshape(1, _H)
    bo = out_proj_b.reshape(1, _H)

    nblk = _B // _BLK
    logits, idx, w, lv, ent = pl.pallas_call(
        _main_body,
        grid=(nblk + 1,),
        in_specs=[
            pl.BlockSpec((_BLK, _H), lambda i: (jnp.minimum(i, nblk - 1), 0)),
            pl.BlockSpec((_H, _H), lambda i: (0, 0)),
            pl.BlockSpec((_H, _H), lambda i: (0, 0)),
            pl.BlockSpec((_H, _E), lambda i: (0, 0)),
            pl.BlockSpec((_H, _E), lambda i: (0, 0)),
            pl.BlockSpec((1, _H), lambda i: (0, 0)),
            pl.BlockSpec((1, _H), lambda i: (0, 0)),
        ],
        out_specs=[
            pl.BlockSpec((_BLK, _E), lambda i: (jnp.minimum(i, nblk - 1), 0)),
            pl.BlockSpec((_BLK, 2), lambda i: (jnp.maximum(i - 1, 0), 0)),
            pl.BlockSpec((_BLK, 2), lambda i: (jnp.maximum(i - 1, 0), 0)),
            pl.BlockSpec((1, 1), lambda i: (0, 0)),
            pl.BlockSpec((1, 1), lambda i: (0, 0)),
        ],
        out_shape=[
            jax.ShapeDtypeStruct((_B, _E), jnp.float32),
            jax.ShapeDtypeStruct((_B, 2), jnp.int32),
            jax.ShapeDtypeStruct((_B, 2), jnp.float32),
            jax.ShapeDtypeStruct((1, 1), jnp.float32),
            jax.ShapeDtypeStruct((1, 1), jnp.float32),
        ],
        scratch_shapes=[
            pltpu.VMEM((_BLK, _E), jnp.float32),
            pltpu.VMEM((1, _E), jnp.float32),
            pltpu.VMEM((1, 1), jnp.float32),
        ],
        compiler_params=pltpu.CompilerParams(
            dimension_semantics=("arbitrary",),
            vmem_limit_bytes=100 * 1024 * 1024),
    )(hidden_states, wvt, wot, rw1t, rw2t, bv, bo)

    return (logits, idx, w, lv.reshape(()), ent.reshape(()))


# speedup vs baseline: 1.1228x; 1.0285x over previous
import jax
import jax.numpy as jnp
from jax import lax
from jax.experimental import pallas as pl
from jax.experimental.pallas import tpu as pltpu

_H = 768
_E = 64
_B = 32768
_BLK = 2048


def _dotnt(a, b):
    return lax.dot_general(a, b, (((1,), (1,)), ((), ())),
                           preferred_element_type=jnp.float32)


def _main_body(x_ref, wv_ref, wo_ref, rw1_ref, rw2_ref, bv_ref, bo_ref,
               logits_ref, idx_ref, w_ref, lv_ref, ent_ref,
               ls_ref, load_acc, ent_acc,
               wv16_ref, wo16_ref, rw116_ref, rw216_ref):
    i = pl.program_id(0)
    nblk = pl.num_programs(0) - 1
    first = i == 0

    @pl.when(first)
    def _prep_weights():
        wv16_ref[...] = wv_ref[0].astype(jnp.bfloat16)
        wo16_ref[...] = wo_ref[...].astype(jnp.bfloat16)
        rw116_ref[...] = rw1_ref[...].astype(jnp.bfloat16)
        rw216_ref[...] = rw2_ref[...].astype(jnp.bfloat16)

    l = ls_ref[...]
    m1 = jnp.max(l, axis=-1, keepdims=True)
    d = l - m1
    e = jnp.exp(d)
    s = jnp.sum(e, axis=-1, keepdims=True)
    rs = 1.0 / s
    p = e * rs

    load_c = jnp.where(first, 0.0, jnp.sum(p, axis=0, keepdims=True))
    load_acc[...] = jnp.where(first, 0.0, load_acc[...]) + load_c
    plogp = jnp.sum(e * d, axis=-1, keepdims=True) * rs - jnp.log(s)
    ent_c = jnp.where(first, 0.0, jnp.sum(plogp, axis=0, keepdims=True))
    ent_acc[...] = jnp.where(first, 0.0, ent_acc[...]) + ent_c

    colf = lax.broadcasted_iota(jnp.int32, l.shape, 1).astype(jnp.float32)
    big = jnp.float32(_E)
    i1f = jnp.min(jnp.where(l == m1, colf, big), axis=-1, keepdims=True)
    masked = jnp.where(colf == i1f, -jnp.inf, l)
    m2 = jnp.max(masked, axis=-1, keepdims=True)
    i2f = jnp.min(jnp.where(masked == m2, colf, big), axis=-1, keepdims=True)
    idx_ref[...] = jnp.concatenate([i1f, i2f], axis=1).astype(jnp.int32)

    t = jnp.exp(m2 - m1)
    w1 = 1.0 / (1.0 + t)
    w_ref[...] = jnp.concatenate([w1, 1.0 - w1], axis=1)

    x16 = x_ref[...].astype(jnp.bfloat16)
    v = _dotnt(x16, wv16_ref[...]) + bv_ref[0]
    a = _dotnt(v.astype(jnp.bfloat16), wo16_ref[...]) + bo_ref[...]
    logits = _dotnt(x16, rw116_ref[...]) + _dotnt(a.astype(jnp.bfloat16),
                                                  rw216_ref[...])
    logits_ref[...] = logits
    ls_ref[...] = logits

    @pl.when(i == nblk)
    def _finalize():
        el = load_acc[...] / _B
        mu = jnp.mean(el)
        lv_ref[...] = jnp.sum((el - mu) ** 2, keepdims=True)[:, :1] / (_E - 1)
        ent_ref[...] = -ent_acc[...] / _B


def kernel(hidden_states, enc_w1, enc_b1, ln_g, ln_b, enc_w2, enc_b2,
           in_proj_w, in_proj_b, out_proj_w, out_proj_b, router_w):
    ipw3 = in_proj_w.reshape(3, _H, _H)
    ipb3 = in_proj_b.reshape(3, 1, _H)
    bo2 = out_proj_b.reshape(1, _H)

    nblk = _B // _BLK
    logits, idx, w, lv, ent = pl.pallas_call(
        _main_body,
        grid=(nblk + 1,),
        in_specs=[
            pl.BlockSpec((_BLK, _H), lambda i: (jnp.minimum(i, nblk - 1), 0)),
            pl.BlockSpec((1, _H, _H), lambda i: (2, 0, 0)),
            pl.BlockSpec((_H, _H), lambda i: (0, 0)),
            pl.BlockSpec((_E, _H), lambda i: (0, 0)),
            pl.BlockSpec((_E, _H), lambda i: (0, 1)),
            pl.BlockSpec((1, 1, _H), lambda i: (2, 0, 0)),
            pl.BlockSpec((1, _H), lambda i: (0, 0)),
        ],
        out_specs=[
            pl.BlockSpec((_BLK, _E), lambda i: (jnp.minimum(i, nblk - 1), 0)),
            pl.BlockSpec((_BLK, 2), lambda i: (jnp.maximum(i - 1, 0), 0)),
            pl.BlockSpec((_BLK, 2), lambda i: (jnp.maximum(i - 1, 0), 0)),
            pl.BlockSpec((1, 1), lambda i: (0, 0)),
            pl.BlockSpec((1, 1), lambda i: (0, 0)),
        ],
        out_shape=[
            jax.ShapeDtypeStruct((_B, _E), jnp.float32),
            jax.ShapeDtypeStruct((_B, 2), jnp.int32),
            jax.ShapeDtypeStruct((_B, 2), jnp.float32),
            jax.ShapeDtypeStruct((1, 1), jnp.float32),
            jax.ShapeDtypeStruct((1, 1), jnp.float32),
        ],
        scratch_shapes=[
            pltpu.VMEM((_BLK, _E), jnp.float32),
            pltpu.VMEM((1, _E), jnp.float32),
            pltpu.VMEM((1, 1), jnp.float32),
            pltpu.VMEM((_H, _H), jnp.bfloat16),
            pltpu.VMEM((_H, _H), jnp.bfloat16),
            pltpu.VMEM((_E, _H), jnp.bfloat16),
            pltpu.VMEM((_E, _H), jnp.bfloat16),
        ],
        compiler_params=pltpu.CompilerParams(
            dimension_semantics=("arbitrary",)),
    )(hidden_states, ipw3, out_proj_w, router_w, router_w, ipb3, bo2)

    return (logits, idx, w, lv.reshape(()), ent.reshape(()))
